# baseline (speedup 1.0000x reference)
import functools

import jax
import jax.numpy as jnp
from jax import lax
from jax.experimental import pallas as pl
from jax.experimental.pallas import tpu as pltpu
from jax.experimental.pallas import tpu_sc as plsc

_NC = 2
_NS = 16
_NW = _NC * _NS

_N = 16384 * 200
_PER_W = _N // _NW
_CHUNK = 10240
_NCHUNK = _PER_W // _CHUNK
_HBM_CHUNK = 2
_SEQ = [i for i in range(_NCHUNK) if i != _HBM_CHUNK]

_V = 1000000
_NST = 6
_STMAIN = _NST * _NS * _CHUNK
_VTAIL = (_V - _STMAIN) // 8


def _sc_gather(table, idx_flat):
    mesh = plsc.VectorSubcoreMesh(core_axis_name="c", subcore_axis_name="s")

    scratch = (
        [pltpu.VMEM_SHARED((_V,), jnp.int32)]
        + [pltpu.VMEM((_CHUNK,), jnp.int32) for _ in range(6)]
        + [pltpu.SemaphoreType.DMA for _ in range(11)]
    )

    @functools.partial(
        pl.kernel,
        mesh=mesh,
        out_type=jax.ShapeDtypeStruct((_N,), jnp.int32),
        scratch_types=scratch,
    )
    def k(table_hbm, idx_hbm, out_hbm, tab_s, *refs):
        idx_v = refs[0:3]
        val_v = refs[3:6]
        s_in = refs[6:9]
        s_g = refs[9:12]
        s_o = refs[12:15]
        s_sta = refs[15]
        s_stb = refs[16]

        cid = lax.axis_index("c")
        sid = lax.axis_index("s")
        wid = sid * _NC + cid
        base = wid * _PER_W

        def off(i):
            return base + i * _CHUNK

        in_d = {}
        g_d = {}
        o_d = {}

        in_d[_SEQ[0]] = pltpu.async_copy(
            idx_hbm.at[pl.ds(off(_SEQ[0]), _CHUNK)], idx_v[0], s_in[0])
        in_d[_SEQ[1]] = pltpu.async_copy(
            idx_hbm.at[pl.ds(off(_SEQ[1]), _CHUNK)], idx_v[1], s_in[1])
        in_d[_HBM_CHUNK] = pltpu.async_copy(
            idx_hbm.at[pl.ds(off(_HBM_CHUNK), _CHUNK)], idx_v[2], s_in[2])

        in_d[_HBM_CHUNK].wait()
        g_d[_HBM_CHUNK] = pltpu.async_copy(
            table_hbm.at[idx_v[2]], val_v[2], s_g[2])

        bounce = (val_v[0], val_v[1])
        s_bin = (s_g[0], s_g[1])
        s_bout = (s_sta, s_stb)

        def st_off(t):
            return (t * _NS + sid) * _CHUNK

        st_in = [None] * _NST
        st_out = [None] * _NST
        for t in range(2):
            st_in[t] = pltpu.async_copy(
                table_hbm.at[pl.ds(st_off(t), _CHUNK)], bounce[t], s_bin[t])
        for t in range(_NST):
            b = t % 2
            st_in[t].wait()
            st_out[t] = pltpu.async_copy(
                bounce[b], tab_s.at[pl.ds(st_off(t), _CHUNK)], s_bout[b])
            if t + 2 < _NST:
                st_out[t].wait()
                st_in[t + 2] = pltpu.async_copy(
                    table_hbm.at[pl.ds(st_off(t + 2), _CHUNK)], bounce[b],
                    s_bin[b])
        st_out[_NST - 2].wait()
        st_out[_NST - 1].wait()

        @pl.when(sid < 8)
        def _tail():
            toff = _STMAIN + sid * _VTAIL
            pltpu.async_copy(
                table_hbm.at[pl.ds(toff, _VTAIL)],
                bounce[0].at[pl.ds(0, _VTAIL)], s_bin[0]).wait()
            pltpu.async_copy(
                bounce[0].at[pl.ds(0, _VTAIL)],
                tab_s.at[pl.ds(toff, _VTAIL)], s_bout[0]).wait()

        plsc.subcore_barrier()

        ns = len(_SEQ)
        in_d[_SEQ[0]].wait()
        g_d[_SEQ[0]] = pltpu.async_copy(
            tab_s.at[idx_v[0]], val_v[0], s_g[0])
        for p in range(ns):
            i = _SEQ[p]
            b = p % 2
            if p + 1 < ns:
                nxt = _SEQ[p + 1]
                nb = (p + 1) % 2
                in_d[nxt].wait()
                if p + 1 >= 2:
                    o_d[_SEQ[p - 1]].wait()
                g_d[nxt] = pltpu.async_copy(
                    tab_s.at[idx_v[nb]], val_v[nb], s_g[nb])
            g_d[i].wait()
            o_d[i] = pltpu.async_copy(
                val_v[b], out_hbm.at[pl.ds(off(i), _CHUNK)], s_o[b])
            if p + 2 < ns:
                nn = _SEQ[p + 2]
                in_d[nn] = pltpu.async_copy(
                    idx_hbm.at[pl.ds(off(nn), _CHUNK)], idx_v[b], s_in[b])

        g_d[_HBM_CHUNK].wait()
        o_d[_HBM_CHUNK] = pltpu.async_copy(
            val_v[2], out_hbm.at[pl.ds(off(_HBM_CHUNK), _CHUNK)], s_o[2])

        for i in (_SEQ[-2], _SEQ[-1], _HBM_CHUNK):
            o_d[i].wait()

    return k(table, idx_flat)


def kernel(inputs, table):
    raw = (inputs.T.reshape(25, 8, 128, 128)
           .transpose(0, 2, 1, 3).reshape(-1))
    out_raw = _sc_gather(table, raw)
    out_t = (out_raw.reshape(25, 128, 8, 128)
             .transpose(0, 2, 1, 3).reshape(200, 16384))
    return out_t.T


# speedup vs baseline: 1.1211x; 1.1211x over previous
import functools

import jax
import jax.numpy as jnp
from jax import lax
from jax.experimental import pallas as pl
from jax.experimental.pallas import tpu as pltpu
from jax.experimental.pallas import tpu_sc as plsc

_NC = 2
_NS = 16
_NW = _NC * _NS

_N = 16384 * 200
_PER_W = _N // _NW
_CHUNK = 10240
_NCHUNK = _PER_W // _CHUNK
_B = 3

_V = 1000000
_NST = 6
_STMAIN = _NST * _NS * _CHUNK
_VTAIL = (_V - _STMAIN) // 8


def _sc_gather(table, idx_flat):
    mesh = plsc.VectorSubcoreMesh(core_axis_name="c", subcore_axis_name="s")

    scratch = (
        [pltpu.VMEM_SHARED((_V,), jnp.int32)]
        + [pltpu.VMEM((_CHUNK,), jnp.int32) for _ in range(2 * _B)]
        + [pltpu.SemaphoreType.DMA for _ in range(3 * _B + 3)]
    )

    @functools.partial(
        pl.kernel,
        mesh=mesh,
        out_type=jax.ShapeDtypeStruct((_N,), jnp.int32),
        scratch_types=scratch,
    )
    def k(table_hbm, idx_hbm, out_hbm, tab_s, *refs):
        idx_v = refs[0:_B]
        val_v = refs[_B:2 * _B]
        s_in = refs[2 * _B:3 * _B]
        s_g = refs[3 * _B:4 * _B]
        s_o = refs[4 * _B:5 * _B]
        s_sta = refs[5 * _B]
        s_stb = refs[5 * _B + 1]
        s_stc = refs[5 * _B + 2]

        cid = lax.axis_index("c")
        sid = lax.axis_index("s")
        wid = sid * _NC + cid
        base = wid * _PER_W

        def off(i):
            return base + i * _CHUNK

        in_d = [None] * _NCHUNK
        g_d = [None] * _NCHUNK
        o_d = [None] * _NCHUNK

        for i in range(min(_B, _NCHUNK)):
            in_d[i] = pltpu.async_copy(
                idx_hbm.at[pl.ds(off(i), _CHUNK)], idx_v[i % _B], s_in[i % _B])

        bounce = (val_v[0], val_v[1], val_v[2])
        s_bin = (s_g[0], s_g[1], s_g[2])
        s_bout = (s_sta, s_stb, s_stc)

        def st_off(t):
            return (t * _NS + sid) * _CHUNK

        st_in = [None] * _NST
        st_out = [None] * _NST
        for t in range(3):
            st_in[t] = pltpu.async_copy(
                table_hbm.at[pl.ds(st_off(t), _CHUNK)], bounce[t], s_bin[t])
        for t in range(_NST):
            b = t % 3
            st_in[t].wait()
            st_out[t] = pltpu.async_copy(
                bounce[b], tab_s.at[pl.ds(st_off(t), _CHUNK)], s_bout[b])
            if t + 3 < _NST:
                st_out[t].wait()
                st_in[t + 3] = pltpu.async_copy(
                    table_hbm.at[pl.ds(st_off(t + 3), _CHUNK)], bounce[b],
                    s_bin[b])
        for t in range(_NST - 3, _NST):
            st_out[t].wait()

        @pl.when(sid < 8)
        def _tail():
            toff = _STMAIN + sid * _VTAIL
            pltpu.async_copy(
                table_hbm.at[pl.ds(toff, _VTAIL)],
                bounce[0].at[pl.ds(0, _VTAIL)], s_bin[0]).wait()
            pltpu.async_copy(
                bounce[0].at[pl.ds(0, _VTAIL)],
                tab_s.at[pl.ds(toff, _VTAIL)], s_bout[0]).wait()

        plsc.subcore_barrier()

        for j in range(2):
            in_d[j].wait()
            g_d[j] = pltpu.async_copy(tab_s.at[idx_v[j]], val_v[j], s_g[j])
        for i in range(_NCHUNK):
            b = i % _B
            if i + 2 < _NCHUNK:
                nb = (i + 2) % _B
                in_d[i + 2].wait()
                if i + 2 >= _B:
                    o_d[i + 2 - _B].wait()
                g_d[i + 2] = pltpu.async_copy(
                    tab_s.at[idx_v[nb]], val_v[nb], s_g[nb])
            g_d[i].wait()
            o_d[i] = pltpu.async_copy(
                val_v[b], out_hbm.at[pl.ds(off(i), _CHUNK)], s_o[b])
            if i + _B < _NCHUNK:
                in_d[i + _B] = pltpu.async_copy(
                    idx_hbm.at[pl.ds(off(i + _B), _CHUNK)], idx_v[b], s_in[b])

        for i in range(max(0, _NCHUNK - _B), _NCHUNK):
            o_d[i].wait()

    return k(table, idx_flat)


def kernel(inputs, table):
    raw = (inputs.T.reshape(25, 8, 128, 128)
           .transpose(0, 2, 1, 3).reshape(-1))
    out_raw = _sc_gather(table, raw)
    out_t = (out_raw.reshape(25, 128, 8, 128)
             .transpose(0, 2, 1, 3).reshape(200, 16384))
    return out_t.T
